# baseline (speedup 1.0000x reference)
import jax
import jax.numpy as jnp
from jax import lax
from jax.experimental import pallas as pl
from jax.experimental.pallas import tpu as pltpu

_K = 128
_N = 32768
_RPB = 32


def _body(x_ref, o_ref, key_ref, j_ref, t_ref, cg_ref, found_ref, rem_ref,
          ub_ref, cons_ref):
    minint = jnp.int32(-2147483648)
    kf = jnp.float32(_K)

    x = x_ref[...]
    s = lax.bitcast_convert_type(x, jnp.int32)
    key = jnp.where(s >= 0, s, minint - (s + jnp.int32(1)))
    key_ref[...] = key

    def count_ge(th):
        m = (key >= th).astype(jnp.int32)
        sl = _N // 8
        parts = [jnp.sum(m[:, i * sl:(i + 1) * sl], axis=1,
                         keepdims=True) for i in range(8)]
        return sum(parts[1:], parts[0]).astype(jnp.float32)

    def count_eq(v):
        m = (key == v).astype(jnp.int32)
        sl = _N // 8
        parts = [jnp.sum(m[:, i * sl:(i + 1) * sl], axis=1,
                         keepdims=True) for i in range(8)]
        return sum(parts[1:], parts[0]).astype(jnp.float32)

    def masked_max(lo_b, ub_b):
        mv = jnp.where((key >= lo_b) & (key <= ub_b), key, minint)
        sl = _N // 8
        parts = [jnp.max(mv[:, i * sl:(i + 1) * sl], axis=1,
                         keepdims=True) for i in range(8)]
        out = parts[0]
        for p in parts[1:]:
            out = jnp.maximum(out, p)
        return out

    cnt0 = count_ge(jnp.zeros((_RPB, 1), jnp.int32))
    ge0 = cnt0 >= kf
    lo = jnp.where(ge0, jnp.int32(0), minint)
    hi = jnp.where(ge0, jnp.int32(2147483647), jnp.int32(-1))
    cnt_lo = jnp.where(ge0, cnt0, jnp.float32(_N))
    cnt_hi1 = jnp.where(ge0, jnp.float32(0.0), cnt0)

    def step(_, carry):
        lo, hi, cnt_lo, cnt_hi1 = carry
        d = hi - lo
        mid = lo + (d >> 1) + (d & 1)
        c = count_ge(mid)
        ge = c >= kf
        return (jnp.where(ge, mid, lo),
                jnp.where(ge, hi, mid - 1),
                jnp.where(ge, c, cnt_lo),
                jnp.where(ge, cnt_hi1, c))

    lo, hi, cnt_lo, cnt_hi1 = lax.fori_loop(
        0, 21, step, (lo, hi, cnt_lo, cnt_hi1))

    n_in = cnt_lo - cnt_hi1
    t_ref[...] = lo
    cg_ref[...] = cnt_lo

    @pl.when(jnp.all(n_in <= jnp.float32(4.0)))
    def _finish_by_max():
        found_ref[...] = jnp.zeros((_RPB, 1), jnp.int32)
        rem_ref[...] = kf - cnt_hi1
        ub_ref[...] = hi
        cons_ref[...] = cnt_hi1

        for _r in range(4):
            @pl.when(jnp.any(found_ref[...] == 0))
            def _round():
                found = found_ref[...] != 0
                mval = masked_max(lo, ub_ref[...])
                cm = count_eq(mval)
                hit = (~found) & (rem_ref[...] <= cm)
                t_ref[...] = jnp.where(hit, mval, t_ref[...])
                cg_ref[...] = jnp.where(hit, cons_ref[...] + cm,
                                        cg_ref[...])
                cons_ref[...] = jnp.where(found, cons_ref[...],
                                          cons_ref[...] + cm)
                rem_ref[...] = jnp.where(found, rem_ref[...],
                                         rem_ref[...] - cm)
                ub_ref[...] = jnp.where(found, ub_ref[...],
                                        mval - jnp.int32(1))
                found_ref[...] = (found | hit).astype(jnp.int32)

    @pl.when(jnp.any(n_in > jnp.float32(4.0)))
    def _finish_by_bisect():
        flo, fhi, fcnt, _ = lax.fori_loop(
            0, 10, step, (lo, hi, cnt_lo, cnt_hi1))
        t_ref[...] = flo
        cg_ref[...] = fcnt

    t = t_ref[...]
    cnt_ge = cg_ref[...]

    j_ref[...] = jnp.full((_RPB, 1), jnp.int32(_N - 1))

    col = lax.broadcasted_iota(jnp.int32, (_RPB, _N), 1)

    @pl.when(jnp.any(cnt_ge > kf))
    def _resolve_ties():
        eq = key == t
        cnt_eq = jnp.sum(eq.astype(jnp.float32), axis=1, keepdims=True)
        m = (kf - (cnt_ge - cnt_eq)).astype(jnp.int32)

        def jstep(_, carry):
            jlo, jhi = carry
            mid = (jlo + jhi) >> 1
            c = jnp.sum((eq & (col <= mid)).astype(jnp.float32),
                        axis=1, keepdims=True)
            p = c.astype(jnp.int32) >= m
            return jnp.where(p, jlo, mid + 1), jnp.where(p, mid, jhi)

        jlo, _ = lax.fori_loop(
            0, 15, jstep,
            (jnp.zeros((_RPB, 1), jnp.int32),
             jnp.full((_RPB, 1), jnp.int32(_N - 1))))
        j_ref[...] = jlo

    j = j_ref[...]
    tpos = jnp.maximum(t, jnp.int32(0))
    mask = (key > tpos) | ((key == tpos) & (t > 0) & (col <= j))
    xv = lax.bitcast_convert_type(key, jnp.float32)
    o_ref[...] = jnp.where(mask, xv, jnp.float32(0.0))


def kernel(x):
    rows = x.shape[0]
    return pl.pallas_call(
        _body,
        grid=(rows // _RPB,),
        in_specs=[pl.BlockSpec((_RPB, _N), lambda i: (i, 0))],
        out_specs=pl.BlockSpec((_RPB, _N), lambda i: (i, 0)),
        out_shape=jax.ShapeDtypeStruct(x.shape, x.dtype),
        scratch_shapes=[
            pltpu.VMEM((_RPB, _N), jnp.int32),
            pltpu.VMEM((_RPB, 1), jnp.int32),
            pltpu.VMEM((_RPB, 1), jnp.int32),
            pltpu.VMEM((_RPB, 1), jnp.float32),
            pltpu.VMEM((_RPB, 1), jnp.int32),
            pltpu.VMEM((_RPB, 1), jnp.float32),
            pltpu.VMEM((_RPB, 1), jnp.int32),
            pltpu.VMEM((_RPB, 1), jnp.float32),
        ],
    )(x)


# speedup vs baseline: 1.0554x; 1.0554x over previous
import jax
import jax.numpy as jnp
from jax import lax
from jax.experimental import pallas as pl
from jax.experimental.pallas import tpu as pltpu

_K = 128
_N = 32768
_RPB = 64


def _body(x_ref, o_ref, key_ref, j_ref, t_ref, cg_ref, found_ref, rem_ref,
          ub_ref, cons_ref):
    minint = jnp.int32(-2147483648)
    kf = jnp.float32(_K)

    x = x_ref[...]
    s = lax.bitcast_convert_type(x, jnp.int32)
    key = jnp.where(s >= 0, s, minint - (s + jnp.int32(1)))
    key_ref[...] = key

    def count_ge(th):
        m = (key >= th).astype(jnp.int32)
        sl = _N // 8
        parts = [jnp.sum(m[:, i * sl:(i + 1) * sl], axis=1,
                         keepdims=True) for i in range(8)]
        return sum(parts[1:], parts[0]).astype(jnp.float32)

    def count_eq(v):
        m = (key == v).astype(jnp.int32)
        sl = _N // 8
        parts = [jnp.sum(m[:, i * sl:(i + 1) * sl], axis=1,
                         keepdims=True) for i in range(8)]
        return sum(parts[1:], parts[0]).astype(jnp.float32)

    def masked_max(lo_b, ub_b):
        mv = jnp.where((key >= lo_b) & (key <= ub_b), key, minint)
        sl = _N // 8
        parts = [jnp.max(mv[:, i * sl:(i + 1) * sl], axis=1,
                         keepdims=True) for i in range(8)]
        out = parts[0]
        for p in parts[1:]:
            out = jnp.maximum(out, p)
        return out

    cnt0 = count_ge(jnp.zeros((_RPB, 1), jnp.int32))
    ge0 = cnt0 >= kf
    lo = jnp.where(ge0, jnp.int32(0), minint)
    hi = jnp.where(ge0, jnp.int32(2147483647), jnp.int32(-1))
    cnt_lo = jnp.where(ge0, cnt0, jnp.float32(_N))
    cnt_hi1 = jnp.where(ge0, jnp.float32(0.0), cnt0)

    def step(_, carry):
        lo, hi, cnt_lo, cnt_hi1 = carry
        d = hi - lo
        mid = lo + (d >> 1) + (d & 1)
        c = count_ge(mid)
        ge = c >= kf
        return (jnp.where(ge, mid, lo),
                jnp.where(ge, hi, mid - 1),
                jnp.where(ge, c, cnt_lo),
                jnp.where(ge, cnt_hi1, c))

    lo, hi, cnt_lo, cnt_hi1 = lax.fori_loop(
        0, 21, step, (lo, hi, cnt_lo, cnt_hi1))

    n_in = cnt_lo - cnt_hi1
    t_ref[...] = lo
    cg_ref[...] = cnt_lo

    @pl.when(jnp.all(n_in <= jnp.float32(4.0)))
    def _finish_by_max():
        found_ref[...] = jnp.zeros((_RPB, 1), jnp.int32)
        rem_ref[...] = kf - cnt_hi1
        ub_ref[...] = hi
        cons_ref[...] = cnt_hi1

        for _r in range(4):
            @pl.when(jnp.any(found_ref[...] == 0))
            def _round():
                found = found_ref[...] != 0
                mval = masked_max(lo, ub_ref[...])
                cm = count_eq(mval)
                hit = (~found) & (rem_ref[...] <= cm)
                t_ref[...] = jnp.where(hit, mval, t_ref[...])
                cg_ref[...] = jnp.where(hit, cons_ref[...] + cm,
                                        cg_ref[...])
                cons_ref[...] = jnp.where(found, cons_ref[...],
                                          cons_ref[...] + cm)
                rem_ref[...] = jnp.where(found, rem_ref[...],
                                         rem_ref[...] - cm)
                ub_ref[...] = jnp.where(found, ub_ref[...],
                                        mval - jnp.int32(1))
                found_ref[...] = (found | hit).astype(jnp.int32)

    @pl.when(jnp.any(n_in > jnp.float32(4.0)))
    def _finish_by_bisect():
        flo, fhi, fcnt, _ = lax.fori_loop(
            0, 10, step, (lo, hi, cnt_lo, cnt_hi1))
        t_ref[...] = flo
        cg_ref[...] = fcnt

    t = t_ref[...]
    cnt_ge = cg_ref[...]

    j_ref[...] = jnp.full((_RPB, 1), jnp.int32(_N - 1))

    col = lax.broadcasted_iota(jnp.int32, (_RPB, _N), 1)

    @pl.when(jnp.any(cnt_ge > kf))
    def _resolve_ties():
        eq = key == t
        cnt_eq = jnp.sum(eq.astype(jnp.float32), axis=1, keepdims=True)
        m = (kf - (cnt_ge - cnt_eq)).astype(jnp.int32)

        def jstep(_, carry):
            jlo, jhi = carry
            mid = (jlo + jhi) >> 1
            c = jnp.sum((eq & (col <= mid)).astype(jnp.float32),
                        axis=1, keepdims=True)
            p = c.astype(jnp.int32) >= m
            return jnp.where(p, jlo, mid + 1), jnp.where(p, mid, jhi)

        jlo, _ = lax.fori_loop(
            0, 15, jstep,
            (jnp.zeros((_RPB, 1), jnp.int32),
             jnp.full((_RPB, 1), jnp.int32(_N - 1))))
        j_ref[...] = jlo

    j = j_ref[...]
    tpos = jnp.maximum(t, jnp.int32(0))
    mask = (key > tpos) | ((key == tpos) & (t > 0) & (col <= j))
    xv = lax.bitcast_convert_type(key, jnp.float32)
    o_ref[...] = jnp.where(mask, xv, jnp.float32(0.0))


def kernel(x):
    rows = x.shape[0]
    return pl.pallas_call(
        _body,
        grid=(rows // _RPB,),
        in_specs=[pl.BlockSpec((_RPB, _N), lambda i: (i, 0))],
        out_specs=pl.BlockSpec((_RPB, _N), lambda i: (i, 0)),
        out_shape=jax.ShapeDtypeStruct(x.shape, x.dtype),
        scratch_shapes=[
            pltpu.VMEM((_RPB, _N), jnp.int32),
            pltpu.VMEM((_RPB, 1), jnp.int32),
            pltpu.VMEM((_RPB, 1), jnp.int32),
            pltpu.VMEM((_RPB, 1), jnp.float32),
            pltpu.VMEM((_RPB, 1), jnp.int32),
            pltpu.VMEM((_RPB, 1), jnp.float32),
            pltpu.VMEM((_RPB, 1), jnp.int32),
            pltpu.VMEM((_RPB, 1), jnp.float32),
        ],
    )(x)
